# baseline (speedup 1.0000x reference)
import functools

import jax
import jax.numpy as jnp
from jax import lax
from jax.experimental import pallas as pl
from jax.experimental.pallas import tpu as pltpu
from jax.experimental.pallas import tpu_sc as plsc

_N = 10000
_E = 320000
_F = 128
_IN = 512
_ND = 2000
_NT = 8000

_NC = 2
_NS = 16
_NW = _NC * _NS
_FPT = _F // _NW
_WORDS = _FPT * _N

_CHUNK = 8000
_NCHUNK = _E // _CHUNK
_GROUPS = _CHUNK // 16

_HIGH = jax.lax.Precision.DEFAULT
_EXACT = jax.lax.Precision.HIGHEST


def _eye128(dtype=jnp.float32):
    a = lax.broadcasted_iota(jnp.int32, (_F, _F), 0)
    b = lax.broadcasted_iota(jnp.int32, (_F, _F), 1)
    return (a == b).astype(dtype)


def _pack_pairs(s_t):
    sb = s_t.astype(jnp.bfloat16).reshape(_F // 2, 2, s_t.shape[1])
    ev = lax.bitcast_convert_type(sb[:, 0, :], jnp.uint16).astype(jnp.uint32)
    od = lax.bitcast_convert_type(sb[:, 1, :], jnp.uint16).astype(jnp.uint32)
    return lax.bitcast_convert_type(ev | (od << 16), jnp.int32)


def _dense_in_body(x_ref, w_ref, w1_ref, out_ref):
    x = x_ref[...]
    nrm = jnp.sqrt(jnp.sum(x * x, axis=1, keepdims=True))
    xn = x / jnp.maximum(nrm, 1e-12)
    y = jnp.dot(xn, w_ref[...], preferred_element_type=jnp.float32,
                precision=_HIGH)
    s = jnp.dot(y, w1_ref[...], preferred_element_type=jnp.float32,
                precision=_HIGH)
    s_t = lax.dot_general(_eye128(), s, (((1,), (1,)), ((), ())),
                          preferred_element_type=jnp.float32,
                          precision=_EXACT)
    out_ref[...] = _pack_pairs(s_t)


def _dense_in(x, w, w1, rows):
    return pl.pallas_call(
        _dense_in_body,
        out_shape=jax.ShapeDtypeStruct((_F // 2, rows), jnp.int32),
    )(x, w, w1)


def _pack_body(row_ref, col_ref, out_ref):
    out_ref[...] = jnp.bitwise_or(
        jnp.left_shift(row_ref[...], 16), col_ref[...])


def _pack_edges(row, col):
    r2 = row.reshape(_E // 128, 128)
    c2 = col.reshape(_E // 128, 128)
    packed = pl.pallas_call(
        _pack_body,
        out_shape=jax.ShapeDtypeStruct((_E // 128, 128), jnp.int32),
    )(r2, c2)
    return packed.reshape(-1)


def _edge_body(supp_hbm, rc_hbm, ew_hbm, out_hbm,
               supp_v, acc_v,
               rc0_v, ew0_v, rc1_v, ew1_v,
               supp_sem, sr0, sw0, sr1, sw1):
    wid = lax.axis_index("s") * _NC + lax.axis_index("c")
    base = wid * _WORDS
    supp_base = wid * (2 * _N)
    bufs = ((rc0_v, ew0_v), (rc1_v, ew1_v))
    sems = ((sr0, sw0), (sr1, sw1))

    def _edges(ci):
        return (rc_hbm.at[pl.ds(ci * _CHUNK, _CHUNK)],
                ew_hbm.at[pl.ds(ci * _CHUNK, _CHUNK)])

    def _start(ci, b):
        for src, dst, sem in zip(_edges(ci), bufs[b], sems[b]):
            pltpu.async_copy(src, dst, sem)

    def _wait(ci, b):
        for src, dst, sem in zip(_edges(ci), bufs[b], sems[b]):
            pltpu.make_async_copy(src, dst, sem).wait()

    supp_cp = pltpu.async_copy(supp_hbm.at[pl.ds(supp_base, 2 * _N)], supp_v,
                               supp_sem)
    _start(0, 0)

    zero = jnp.zeros((16,), jnp.float32)

    @plsc.parallel_loop(0, _WORDS // 16, 1, unroll=10)
    def _z(i):
        acc_v[pl.ds(i * 16, 16)] = zero

    supp_cp.wait()

    def _compute(b):
        rcv, wv = bufs[b]

        @plsc.parallel_loop(0, _GROUPS, 1, unroll=20)
        def _group(g):
            gb = g * 16
            rc = rcv[pl.ds(gb, 16)]
            w = wv[pl.ds(gb, 16)]
            r = lax.shift_right_logical(rc, 16)
            c = jnp.bitwise_and(rc, 0xFFFF)
            for p in range(_FPT // 2):
                vpk = plsc.load_gather(supp_v, [c + (p * _N)])
                vbf = plsc.bitcast(vpk, jnp.bfloat16)
                lo, hi = plsc.unpack(vbf, format=plsc.PackFormat.INTERLEAVED)
                plsc.addupdate_scatter(acc_v, [r + ((2 * p) * _N)], lo * w)
                plsc.addupdate_scatter(acc_v, [r + ((2 * p + 1) * _N)],
                                       hi * w)

    def _outer(k, carry):
        for b in range(2):
            ci = k * 2 + b
            nxt = ci + 1

            @pl.when(nxt < _NCHUNK)
            def _():
                _start(nxt, 1 - b)

            _wait(ci, b)
            _compute(b)
        return carry

    lax.fori_loop(0, _NCHUNK // 2, _outer, 0)
    pltpu.sync_copy(acc_v, out_hbm.at[pl.ds(base, _WORDS)])


_edge_kernel = functools.partial(
    pl.kernel,
    out_type=jax.ShapeDtypeStruct((_F * _N,), jnp.float32),
    mesh=plsc.VectorSubcoreMesh(core_axis_name="c", subcore_axis_name="s"),
    compiler_params=pltpu.CompilerParams(needs_layout_passes=False),
    scratch_types=[
        pltpu.VMEM((2 * _N,), jnp.int32),
        pltpu.VMEM((_WORDS,), jnp.float32),
        pltpu.VMEM((_CHUNK,), jnp.int32),
        pltpu.VMEM((_CHUNK,), jnp.float32),
        pltpu.VMEM((_CHUNK,), jnp.int32),
        pltpu.VMEM((_CHUNK,), jnp.float32),
        pltpu.SemaphoreType.DMA,
        pltpu.SemaphoreType.DMA,
        pltpu.SemaphoreType.DMA,
        pltpu.SemaphoreType.DMA,
        pltpu.SemaphoreType.DMA,
    ],
)(_edge_body)


def _edge_layer(supp_t_flat, rc, ew):
    return _edge_kernel(supp_t_flat, rc, ew)


def _mid_body(acc_ref, b_ref, w2_ref, out_ref):
    h = jnp.maximum(acc_ref[...] + b_ref[...], 0.0)
    s_t = lax.dot_general(w2_ref[...], h, (((0,), (0,)), ((), ())),
                          preferred_element_type=jnp.float32,
                          precision=_HIGH)
    out_ref[...] = _pack_pairs(s_t)


def _mid_layer(acc_t, b, w2):
    return pl.pallas_call(
        _mid_body,
        out_shape=jax.ShapeDtypeStruct((_F // 2, _N), jnp.int32),
    )(acc_t, b, w2)


def _h_body(acc_ref, b_ref, out_ref):
    h = jnp.maximum(acc_ref[...] + b_ref[...], 0.0)
    out_ref[...] = lax.dot_general(h, _eye128(), (((0,), (0,)), ((), ())),
                                   preferred_element_type=jnp.float32,
                                   precision=_EXACT)


def _h_layer(acc_t, b):
    return pl.pallas_call(
        _h_body,
        out_shape=jax.ShapeDtypeStruct((_N, _F), jnp.float32),
    )(acc_t, b)


def _dec_body(hr_ref, accd_ref, b_ref, tw_ref, out_ref):
    s1 = jnp.dot(hr_ref[...], tw_ref[...],
                 preferred_element_type=jnp.float32,
                 precision=_HIGH)
    hd2_t = jnp.maximum(accd_ref[...] + b_ref[...], 0.0)
    out_ref[...] = jnp.dot(s1, hd2_t,
                           preferred_element_type=jnp.float32,
                           precision=_HIGH)


def _decoder(h, acc2d, b, tw, bi=400):
    return pl.pallas_call(
        _dec_body,
        grid=(_ND // bi,),
        in_specs=[
            pl.BlockSpec((bi, _F), lambda i: (i, 0)),
            pl.BlockSpec((_F, _NT), lambda i: (0, 0)),
            pl.BlockSpec((_F, 1), lambda i: (0, 0)),
            pl.BlockSpec((_F, _F), lambda i: (0, 0)),
        ],
        out_specs=pl.BlockSpec((bi, _NT), lambda i: (i, 0)),
        out_shape=jax.ShapeDtypeStruct((_ND, _NT), jnp.float32),
    )(h, acc2d, b, tw)


def kernel(drug_F, target_F, edge_index, edge_weight, fc1_W, fc2_W,
           W1, b1, W2, b2, train_W, drug_num, target_num):
    row = edge_index[0].astype(jnp.int32)
    col = edge_index[1].astype(jnp.int32)
    ew = edge_weight.astype(jnp.float32)
    b1c = b1.reshape(_F, 1)
    b2c = b2.reshape(_F, 1)

    rc = _pack_edges(row, col)
    supp1_d = _dense_in(drug_F, fc1_W, W1, _ND)
    supp1_t = _dense_in(target_F, fc2_W, W1, _NT)
    supp1 = jnp.concatenate([supp1_d, supp1_t], axis=1)

    acc1 = _edge_layer(supp1.reshape(-1), rc, ew).reshape(_F, _N)
    supp2 = _mid_layer(acc1, b1c, W2)
    acc2 = _edge_layer(supp2.reshape(-1), rc, ew).reshape(_F, _N)

    H = _h_layer(acc2, b2c)
    decoder = _decoder(H, acc2[:, _ND:], b2c, train_W)
    return (decoder, H)


# speedup vs baseline: 1.1911x; 1.1911x over previous
import functools

import jax
import jax.numpy as jnp
from jax import lax
from jax.experimental import pallas as pl
from jax.experimental.pallas import tpu as pltpu
from jax.experimental.pallas import tpu_sc as plsc

_N = 10000
_E = 320000
_F = 128
_IN = 512
_ND = 2000
_NT = 8000

_NC = 2
_NS = 16
_NW = _NC * _NS
_FPT = _F // _NW
_WORDS = _FPT * _N

_CHUNK = 8000
_NCHUNK = _E // _CHUNK
_GROUPS = _CHUNK // 16

_HIGH = jax.lax.Precision.DEFAULT
_EXACT = jax.lax.Precision.HIGHEST


def _eye128(dtype=jnp.float32):
    a = lax.broadcasted_iota(jnp.int32, (_F, _F), 0)
    b = lax.broadcasted_iota(jnp.int32, (_F, _F), 1)
    return (a == b).astype(dtype)


def _pack_pairs(s_t):
    sb = s_t.astype(jnp.bfloat16).reshape(_F // 2, 2, s_t.shape[1])
    ev = lax.bitcast_convert_type(sb[:, 0, :], jnp.uint16).astype(jnp.uint32)
    od = lax.bitcast_convert_type(sb[:, 1, :], jnp.uint16).astype(jnp.uint32)
    return lax.bitcast_convert_type(ev | (od << 16), jnp.int32)


def _dense_in_body(x_ref, w_ref, w1_ref, out_ref):
    x = x_ref[...]
    nrm = jnp.sqrt(jnp.sum(x * x, axis=1, keepdims=True))
    xn = x / jnp.maximum(nrm, 1e-12)
    y = jnp.dot(xn, w_ref[...], preferred_element_type=jnp.float32,
                precision=_HIGH)
    s = jnp.dot(y, w1_ref[...], preferred_element_type=jnp.float32,
                precision=_HIGH)
    s_t = lax.dot_general(_eye128(), s, (((1,), (1,)), ((), ())),
                          preferred_element_type=jnp.float32,
                          precision=_EXACT)
    out_ref[...] = _pack_pairs(s_t)


def _dense_in(x, w, w1, rows):
    return pl.pallas_call(
        _dense_in_body,
        out_shape=jax.ShapeDtypeStruct((_F // 2, rows), jnp.int32),
    )(x, w, w1)


def _pack_body(row_ref, col_ref, out_ref):
    out_ref[...] = jnp.bitwise_or(
        jnp.left_shift(row_ref[...], 16), col_ref[...])


def _pack_edges(row, col):
    r2 = row.reshape(_E // 128, 128)
    c2 = col.reshape(_E // 128, 128)
    packed = pl.pallas_call(
        _pack_body,
        out_shape=jax.ShapeDtypeStruct((_E // 128, 128), jnp.int32),
    )(r2, c2)
    return packed.reshape(-1)


def _edge_body(supp_hbm, rc_hbm, ew_hbm, out_hbm,
               supp_v, acc_v,
               rc0_v, ew0_v, rc1_v, ew1_v,
               supp_sem, sr0, sw0, sr1, sw1):
    wid = lax.axis_index("s") * _NC + lax.axis_index("c")
    base = wid * _WORDS
    supp_base = wid * (2 * _N)
    bufs = ((rc0_v, ew0_v), (rc1_v, ew1_v))
    sems = ((sr0, sw0), (sr1, sw1))

    def _edges(ci):
        return (rc_hbm.at[pl.ds(ci * _CHUNK, _CHUNK)],
                ew_hbm.at[pl.ds(ci * _CHUNK, _CHUNK)])

    def _start(ci, b):
        for src, dst, sem in zip(_edges(ci), bufs[b], sems[b]):
            pltpu.async_copy(src, dst, sem)

    def _wait(ci, b):
        for src, dst, sem in zip(_edges(ci), bufs[b], sems[b]):
            pltpu.make_async_copy(src, dst, sem).wait()

    supp_cp = pltpu.async_copy(supp_hbm.at[pl.ds(supp_base, 2 * _N)], supp_v,
                               supp_sem)
    _start(0, 0)

    zero = jnp.zeros((16,), jnp.float32)

    @plsc.parallel_loop(0, _WORDS // 16, 1, unroll=10)
    def _z(i):
        acc_v[pl.ds(i * 16, 16)] = zero

    supp_cp.wait()

    def _compute(b):
        rcv, wv = bufs[b]

        @plsc.parallel_loop(0, _GROUPS, 1, unroll=5)
        def _group(g):
            gb = g * 16
            rc = rcv[pl.ds(gb, 16)]
            w = wv[pl.ds(gb, 16)]
            r = lax.shift_right_logical(rc, 16)
            c = jnp.bitwise_and(rc, 0xFFFF)
            for p in range(_FPT // 2):
                vpk = plsc.load_gather(supp_v, [c + (p * _N)])
                vbf = plsc.bitcast(vpk, jnp.bfloat16)
                lo, hi = plsc.unpack(vbf, format=plsc.PackFormat.INTERLEAVED)
                plsc.addupdate_scatter(acc_v, [r + ((2 * p) * _N)], lo * w)
                plsc.addupdate_scatter(acc_v, [r + ((2 * p + 1) * _N)],
                                       hi * w)

    def _outer(k, carry):
        for b in range(2):
            ci = k * 2 + b
            nxt = ci + 1

            @pl.when(nxt < _NCHUNK)
            def _():
                _start(nxt, 1 - b)

            _wait(ci, b)
            _compute(b)
        return carry

    lax.fori_loop(0, _NCHUNK // 2, _outer, 0)
    pltpu.sync_copy(acc_v, out_hbm.at[pl.ds(base, _WORDS)])


_edge_kernel = functools.partial(
    pl.kernel,
    out_type=jax.ShapeDtypeStruct((_F * _N,), jnp.float32),
    mesh=plsc.VectorSubcoreMesh(core_axis_name="c", subcore_axis_name="s"),
    compiler_params=pltpu.CompilerParams(needs_layout_passes=False),
    scratch_types=[
        pltpu.VMEM((2 * _N,), jnp.int32),
        pltpu.VMEM((_WORDS,), jnp.float32),
        pltpu.VMEM((_CHUNK,), jnp.int32),
        pltpu.VMEM((_CHUNK,), jnp.float32),
        pltpu.VMEM((_CHUNK,), jnp.int32),
        pltpu.VMEM((_CHUNK,), jnp.float32),
        pltpu.SemaphoreType.DMA,
        pltpu.SemaphoreType.DMA,
        pltpu.SemaphoreType.DMA,
        pltpu.SemaphoreType.DMA,
        pltpu.SemaphoreType.DMA,
    ],
)(_edge_body)


def _edge_layer(supp_t_flat, rc, ew):
    return _edge_kernel(supp_t_flat, rc, ew)


def _mid_body(acc_ref, b_ref, w2_ref, out_ref):
    h = jnp.maximum(acc_ref[...] + b_ref[...], 0.0)
    s_t = lax.dot_general(w2_ref[...], h, (((0,), (0,)), ((), ())),
                          preferred_element_type=jnp.float32,
                          precision=_HIGH)
    out_ref[...] = _pack_pairs(s_t)


def _mid_layer(acc_t, b, w2):
    return pl.pallas_call(
        _mid_body,
        out_shape=jax.ShapeDtypeStruct((_F // 2, _N), jnp.int32),
    )(acc_t, b, w2)


def _h_body(acc_ref, b_ref, out_ref):
    h = jnp.maximum(acc_ref[...] + b_ref[...], 0.0)
    out_ref[...] = lax.dot_general(h, _eye128(), (((0,), (0,)), ((), ())),
                                   preferred_element_type=jnp.float32,
                                   precision=_EXACT)


def _h_layer(acc_t, b):
    return pl.pallas_call(
        _h_body,
        out_shape=jax.ShapeDtypeStruct((_N, _F), jnp.float32),
    )(acc_t, b)


def _dec_body(hr_ref, accd_ref, b_ref, tw_ref, out_ref):
    s1 = jnp.dot(hr_ref[...], tw_ref[...],
                 preferred_element_type=jnp.float32,
                 precision=_HIGH)
    hd2_t = jnp.maximum(accd_ref[...] + b_ref[...], 0.0)
    out_ref[...] = jnp.dot(s1, hd2_t,
                           preferred_element_type=jnp.float32,
                           precision=_HIGH)


def _decoder(h, acc2d, b, tw, bi=400):
    return pl.pallas_call(
        _dec_body,
        grid=(_ND // bi,),
        in_specs=[
            pl.BlockSpec((bi, _F), lambda i: (i, 0)),
            pl.BlockSpec((_F, _NT), lambda i: (0, 0)),
            pl.BlockSpec((_F, 1), lambda i: (0, 0)),
            pl.BlockSpec((_F, _F), lambda i: (0, 0)),
        ],
        out_specs=pl.BlockSpec((bi, _NT), lambda i: (i, 0)),
        out_shape=jax.ShapeDtypeStruct((_ND, _NT), jnp.float32),
    )(h, acc2d, b, tw)


def kernel(drug_F, target_F, edge_index, edge_weight, fc1_W, fc2_W,
           W1, b1, W2, b2, train_W, drug_num, target_num):
    row = edge_index[0].astype(jnp.int32)
    col = edge_index[1].astype(jnp.int32)
    ew = edge_weight.astype(jnp.float32)
    b1c = b1.reshape(_F, 1)
    b2c = b2.reshape(_F, 1)

    rc = _pack_edges(row, col)
    supp1_d = _dense_in(drug_F, fc1_W, W1, _ND)
    supp1_t = _dense_in(target_F, fc2_W, W1, _NT)
    supp1 = jnp.concatenate([supp1_d, supp1_t], axis=1)

    acc1 = _edge_layer(supp1.reshape(-1), rc, ew).reshape(_F, _N)
    supp2 = _mid_layer(acc1, b1c, W2)
    acc2 = _edge_layer(supp2.reshape(-1), rc, ew).reshape(_F, _N)

    H = _h_layer(acc2, b2c)
    decoder = _decoder(H, acc2[:, _ND:], b2c, train_W)
    return (decoder, H)


# speedup vs baseline: 1.1931x; 1.0016x over previous
import functools

import jax
import jax.numpy as jnp
from jax import lax
from jax.experimental import pallas as pl
from jax.experimental.pallas import tpu as pltpu
from jax.experimental.pallas import tpu_sc as plsc

_N = 10000
_E = 320000
_F = 128
_IN = 512
_ND = 2000
_NT = 8000

_NC = 2
_NS = 16
_NW = _NC * _NS
_FPT = _F // _NW
_WORDS = _FPT * _N

_CHUNK = 8000
_NCHUNK = _E // _CHUNK
_GROUPS = _CHUNK // 16

_HIGH = jax.lax.Precision.DEFAULT
_EXACT = jax.lax.Precision.HIGHEST


def _eye128(dtype=jnp.float32):
    a = lax.broadcasted_iota(jnp.int32, (_F, _F), 0)
    b = lax.broadcasted_iota(jnp.int32, (_F, _F), 1)
    return (a == b).astype(dtype)


def _pack_pairs(s_t):
    sb = s_t.astype(jnp.bfloat16).reshape(_F // 2, 2, s_t.shape[1])
    ev = lax.bitcast_convert_type(sb[:, 0, :], jnp.uint16).astype(jnp.uint32)
    od = lax.bitcast_convert_type(sb[:, 1, :], jnp.uint16).astype(jnp.uint32)
    return lax.bitcast_convert_type(ev | (od << 16), jnp.int32)


def _dense_in_body(x_ref, w_ref, w1_ref, out_ref):
    x = x_ref[...]
    nrm = jnp.sqrt(jnp.sum(x * x, axis=1, keepdims=True))
    xn = x / jnp.maximum(nrm, 1e-12)
    y = jnp.dot(xn, w_ref[...], preferred_element_type=jnp.float32,
                precision=_HIGH)
    s = jnp.dot(y, w1_ref[...], preferred_element_type=jnp.float32,
                precision=_HIGH)
    s_t = lax.dot_general(_eye128(), s, (((1,), (1,)), ((), ())),
                          preferred_element_type=jnp.float32,
                          precision=_EXACT)
    out_ref[...] = _pack_pairs(s_t)


def _dense_in(x, w, w1, rows):
    return pl.pallas_call(
        _dense_in_body,
        out_shape=jax.ShapeDtypeStruct((_F // 2, rows), jnp.int32),
    )(x, w, w1)


def _pack_body(row_ref, col_ref, out_ref):
    out_ref[...] = jnp.bitwise_or(
        jnp.left_shift(row_ref[...], 16), col_ref[...])


def _pack_edges(row, col):
    r2 = row.reshape(_E // 128, 128)
    c2 = col.reshape(_E // 128, 128)
    packed = pl.pallas_call(
        _pack_body,
        out_shape=jax.ShapeDtypeStruct((_E // 128, 128), jnp.int32),
    )(r2, c2)
    return packed.reshape(-1)


def _edge_body(supp_hbm, rc_hbm, ew_hbm, out_hbm,
               supp_v, acc_v,
               rc0_v, ew0_v, rc1_v, ew1_v,
               supp_sem, sr0, sw0, sr1, sw1):
    wid = lax.axis_index("s") * _NC + lax.axis_index("c")
    base = wid * _WORDS
    supp_base = wid * (2 * _N)
    bufs = ((rc0_v, ew0_v), (rc1_v, ew1_v))
    sems = ((sr0, sw0), (sr1, sw1))

    def _edges(ci):
        return (rc_hbm.at[pl.ds(ci * _CHUNK, _CHUNK)],
                ew_hbm.at[pl.ds(ci * _CHUNK, _CHUNK)])

    def _start(ci, b):
        for src, dst, sem in zip(_edges(ci), bufs[b], sems[b]):
            pltpu.async_copy(src, dst, sem)

    def _wait(ci, b):
        for src, dst, sem in zip(_edges(ci), bufs[b], sems[b]):
            pltpu.make_async_copy(src, dst, sem).wait()

    supp_cp = pltpu.async_copy(supp_hbm.at[pl.ds(supp_base, 2 * _N)], supp_v,
                               supp_sem)
    _start(0, 0)

    zero = jnp.zeros((16,), jnp.float32)

    @plsc.parallel_loop(0, _WORDS // 16, 1, unroll=10)
    def _z(i):
        acc_v[pl.ds(i * 16, 16)] = zero

    supp_cp.wait()

    def _compute(b):
        rcv, wv = bufs[b]

        @plsc.parallel_loop(0, _GROUPS, 1, unroll=4)
        def _group(g):
            gb = g * 16
            rc = rcv[pl.ds(gb, 16)]
            w = wv[pl.ds(gb, 16)]
            r = lax.shift_right_logical(rc, 16)
            c = jnp.bitwise_and(rc, 0xFFFF)
            for p in range(_FPT // 2):
                vpk = plsc.load_gather(supp_v, [c + (p * _N)])
                vbf = plsc.bitcast(vpk, jnp.bfloat16)
                lo, hi = plsc.unpack(vbf, format=plsc.PackFormat.INTERLEAVED)
                plsc.addupdate_scatter(acc_v, [r + ((2 * p) * _N)], lo * w)
                plsc.addupdate_scatter(acc_v, [r + ((2 * p + 1) * _N)],
                                       hi * w)

    def _outer(k, carry):
        for b in range(2):
            ci = k * 2 + b
            nxt = ci + 1

            @pl.when(nxt < _NCHUNK)
            def _():
                _start(nxt, 1 - b)

            _wait(ci, b)
            _compute(b)
        return carry

    lax.fori_loop(0, _NCHUNK // 2, _outer, 0)
    pltpu.sync_copy(acc_v, out_hbm.at[pl.ds(base, _WORDS)])


_edge_kernel = functools.partial(
    pl.kernel,
    out_type=jax.ShapeDtypeStruct((_F * _N,), jnp.float32),
    mesh=plsc.VectorSubcoreMesh(core_axis_name="c", subcore_axis_name="s"),
    compiler_params=pltpu.CompilerParams(needs_layout_passes=False),
    scratch_types=[
        pltpu.VMEM((2 * _N,), jnp.int32),
        pltpu.VMEM((_WORDS,), jnp.float32),
        pltpu.VMEM((_CHUNK,), jnp.int32),
        pltpu.VMEM((_CHUNK,), jnp.float32),
        pltpu.VMEM((_CHUNK,), jnp.int32),
        pltpu.VMEM((_CHUNK,), jnp.float32),
        pltpu.SemaphoreType.DMA,
        pltpu.SemaphoreType.DMA,
        pltpu.SemaphoreType.DMA,
        pltpu.SemaphoreType.DMA,
        pltpu.SemaphoreType.DMA,
    ],
)(_edge_body)


def _edge_layer(supp_t_flat, rc, ew):
    return _edge_kernel(supp_t_flat, rc, ew)


def _mid_body(acc_ref, b_ref, w2_ref, out_ref):
    h = jnp.maximum(acc_ref[...] + b_ref[...], 0.0)
    s_t = lax.dot_general(w2_ref[...], h, (((0,), (0,)), ((), ())),
                          preferred_element_type=jnp.float32,
                          precision=_HIGH)
    out_ref[...] = _pack_pairs(s_t)


def _mid_layer(acc_t, b, w2):
    return pl.pallas_call(
        _mid_body,
        out_shape=jax.ShapeDtypeStruct((_F // 2, _N), jnp.int32),
    )(acc_t, b, w2)


def _h_body(acc_ref, b_ref, out_ref):
    h = jnp.maximum(acc_ref[...] + b_ref[...], 0.0)
    out_ref[...] = lax.dot_general(h, _eye128(), (((0,), (0,)), ((), ())),
                                   preferred_element_type=jnp.float32,
                                   precision=_EXACT)


def _h_layer(acc_t, b):
    return pl.pallas_call(
        _h_body,
        out_shape=jax.ShapeDtypeStruct((_N, _F), jnp.float32),
    )(acc_t, b)


def _dec_body(hr_ref, accd_ref, b_ref, tw_ref, out_ref):
    s1 = jnp.dot(hr_ref[...], tw_ref[...],
                 preferred_element_type=jnp.float32,
                 precision=_HIGH)
    hd2_t = jnp.maximum(accd_ref[...] + b_ref[...], 0.0)
    out_ref[...] = jnp.dot(s1, hd2_t,
                           preferred_element_type=jnp.float32,
                           precision=_HIGH)


def _decoder(h, acc2d, b, tw, bi=400):
    return pl.pallas_call(
        _dec_body,
        grid=(_ND // bi,),
        in_specs=[
            pl.BlockSpec((bi, _F), lambda i: (i, 0)),
            pl.BlockSpec((_F, _NT), lambda i: (0, 0)),
            pl.BlockSpec((_F, 1), lambda i: (0, 0)),
            pl.BlockSpec((_F, _F), lambda i: (0, 0)),
        ],
        out_specs=pl.BlockSpec((bi, _NT), lambda i: (i, 0)),
        out_shape=jax.ShapeDtypeStruct((_ND, _NT), jnp.float32),
    )(h, acc2d, b, tw)


def kernel(drug_F, target_F, edge_index, edge_weight, fc1_W, fc2_W,
           W1, b1, W2, b2, train_W, drug_num, target_num):
    row = edge_index[0].astype(jnp.int32)
    col = edge_index[1].astype(jnp.int32)
    ew = edge_weight.astype(jnp.float32)
    b1c = b1.reshape(_F, 1)
    b2c = b2.reshape(_F, 1)

    rc = _pack_edges(row, col)
    supp1_d = _dense_in(drug_F, fc1_W, W1, _ND)
    supp1_t = _dense_in(target_F, fc2_W, W1, _NT)
    supp1 = jnp.concatenate([supp1_d, supp1_t], axis=1)

    acc1 = _edge_layer(supp1.reshape(-1), rc, ew).reshape(_F, _N)
    supp2 = _mid_layer(acc1, b1c, W2)
    acc2 = _edge_layer(supp2.reshape(-1), rc, ew).reshape(_F, _N)

    H = _h_layer(acc2, b2c)
    decoder = _decoder(H, acc2[:, _ND:], b2c, train_W)
    return (decoder, H)
